# Initial kernel scaffold; baseline (speedup 1.0000x reference)
import jax
import jax.numpy as jnp
from jax.experimental import pallas as pl


def kernel(x, params, edge_index):
    raise NotImplementedError("write your pallas kernel here")


# speedup vs baseline: 7.6326x; 7.6326x over previous
import functools

import jax
import jax.numpy as jnp
import numpy as np
from jax.experimental import pallas as pl

_N = 12586
_E = 201376
_D_IN = 512
_HID = 128
_NL = 6
_B = 62
_S = 203
_HEADS = 8
_HD = 16


def _gelu(x):
    return jax.nn.gelu(x, approximate=False)


def _ln(x, g, b, eps=1e-5):
    m = x.mean(-1, keepdims=True)
    v = ((x - m) ** 2).mean(-1, keepdims=True)
    return (x - m) / jnp.sqrt(v + eps) * g + b


def _pool_weights(levels=(1, 2, 4, 8), s=_S):
    ws = []
    for lvl in levels:
        ps = s // lvl
        w = np.zeros((s,), np.float64)
        for i in range(ps):
            st = (i * s) // ps
            e = -(-((i + 1) * s) // ps)
            w[st:e] += 1.0 / (e - st) / ps
        ws.append(w)
    return np.stack(ws, 0).astype(np.float32)


def kernel(x, params, edge_index):
    p = params
    row, col = edge_index[0], edge_index[1]
    x = _gelu(_ln(x @ p['in_w'].T + p['in_b'], p['in_g'], p['in_bt']))
    for lp in p['layers']:
        idn = x
        h = _ln(x, lp['pre_g'], lp['pre_b'])
        hr = h @ lp['em_w1'][:, :_HID].T
        hc = h @ lp['em_w1'][:, _HID:].T
        xw_gcn = h @ lp['gcn_w'].T
        xw_gat = h @ lp['gat_w'].T
        xg3 = xw_gat.reshape(_N, _HEADS, _HD)
        a_s = (xg3 * lp['gat_as'][None]).sum(-1)
        a_d = (xg3 * lp['gat_ad'][None]).sum(-1)
        mh = a_s.max(0) + a_d.max(0)
        u = hr[row] + hc[col] + lp['em_b1']
        ew = jax.nn.sigmoid(_gelu(u) @ lp['em_w2'][0] + lp['em_b2'][0])
        alpha = jax.nn.leaky_relu(a_s[row] + a_d[col], negative_slope=0.2)
        ex = jnp.exp(alpha - mh)
        deg = jax.ops.segment_sum(ew, col, num_segments=_N) + 1.0
        den = jax.ops.segment_sum(ex, col, num_segments=_N)
        ex_self = jnp.exp(jax.nn.leaky_relu(a_s + a_d, negative_slope=0.2) - mh)
        den = den + ex_self
        dinv = deg ** -0.5
        z = dinv[:, None] * xw_gcn
        accg = jax.ops.segment_sum(ew[:, None] * z[row], col, num_segments=_N)
        acca = jax.ops.segment_sum(
            (ex[:, :, None] * xg3[row]).reshape(_E, _HID), col, num_segments=_N)
        gcn = dinv[:, None] * accg + dinv[:, None] ** 2 * xw_gcn + lp['gcn_b']
        num = acca.reshape(_N, _HEADS, _HD) + ex_self[:, :, None] * xg3
        gat = (num / (den[:, :, None] + 1e-16)).reshape(_N, _HID) + lp['gat_b']
        x = gcn + gat + idn
        x3 = x.reshape(_B, _S, _HID)
        idn3 = x3
        h3 = _ln(x3, lp['post_g'], lp['post_b'])
        h3 = _gelu(h3 @ lp['ffn_w1'].T + lp['ffn_b1']) @ lp['ffn_w2'].T + lp['ffn_b2']
        x3 = h3 + idn3
        y = x3.mean(axis=1)
        sc = jax.nn.sigmoid(_gelu(y @ lp['se_w1'].T) @ lp['se_w2'].T)
        x3 = x3 * sc[:, None, :]
        x = x3.reshape(-1, _HID)
    x3 = x.reshape(_B, _S, _HID)
    pw = jnp.asarray(_pool_weights())
    pooled = jnp.einsum('ls,bsh->blh', pw, x3)
    feats = [pooled[:, i] @ w.T + b for i, (w, b) in enumerate(p['pp'])]
    pyr = jnp.concatenate(feats, axis=-1)
    gf = jnp.concatenate([x3.mean(axis=1), x3.max(axis=1)], axis=-1)
    cat = jnp.concatenate([pyr, gf], axis=-1)
    return _gelu(_ln(cat @ p['fus_w'].T + p['fus_b'], p['fus_g'], p['fus_bt']))


# speedup vs baseline: 22.6238x; 2.9641x over previous
import functools

import jax
import jax.numpy as jnp
import numpy as np
from jax import lax
from jax.experimental import pallas as pl
from jax.experimental.pallas import tpu as pltpu
from jax.experimental.pallas import tpu_sc as plsc

_N = 12586
_E = 201376
_D_IN = 512
_HID = 128
_NL = 6
_B = 62
_S = 203
_HEADS = 8
_HD = 16

_NC, _NS = 2, 16
_NW = _NC * _NS
_N_ACC = 12800
_STRIPE = _N_ACC // _NW
_E_PAD = 204800
_EPW = _E_PAD // _NW
_G = 128


def _gelu(x):
    return 0.5 * x * (1.0 + lax.erf(x * np.float32(0.7071067811865476)))


def _ln(x, g, b, eps=1e-5):
    m = x.mean(-1, keepdims=True)
    v = ((x - m) ** 2).mean(-1, keepdims=True)
    return (x - m) / jnp.sqrt(v + eps) * g + b


def _mesh():
    return plsc.VectorSubcoreMesh(core_axis_name="c", subcore_axis_name="s",
                                  num_cores=_NC, num_subcores=_NS)


def _sc_gather(tabS, tabD, rowp, colp):
    @functools.partial(
        pl.kernel,
        out_type=(jax.ShapeDtypeStruct((_E_PAD, _HID), jnp.float32),
                  jax.ShapeDtypeStruct((_E_PAD, _HID), jnp.float32)),
        mesh=_mesh(),
        scratch_types=[pltpu.VMEM((_G,), jnp.int32),
                       pltpu.VMEM((_G, _HID), jnp.float32),
                       pltpu.SemaphoreType.DMA],
    )
    def k(tabS_h, tabD_h, rowp_h, colp_h, efS_h, efD_h, idx_v, buf_v, sem):
        c = lax.axis_index("c")
        s = lax.axis_index("s")
        epw = _E_PAD // _NS
        base0 = s * epw

        def do(tab_h, idx_h, out_h):
            def body(g, _):
                b = base0 + g * _G
                pltpu.sync_copy(idx_h.at[pl.ds(b, _G)], idx_v)
                pltpu.async_copy(tab_h.at[idx_v], buf_v, sem).wait()
                pltpu.sync_copy(buf_v, out_h.at[pl.ds(b, _G)])
                return _
            lax.fori_loop(0, epw // _G, body, 0)

        @pl.when(c == 0)
        def _():
            do(tabS_h, rowp_h, efS_h)

        @pl.when(c == 1)
        def _():
            do(tabD_h, colp_h, efD_h)

    return k(tabS, tabD, rowp, colp)


def _sc_degden(ewex, colp):
    @functools.partial(
        pl.kernel,
        out_type=jax.ShapeDtypeStruct((_NC, _N_ACC, 16), jnp.float32),
        mesh=_mesh(),
        scratch_types=[pltpu.VMEM((_G,), jnp.int32),
                       pltpu.VMEM((_G, 16), jnp.float32),
                       pltpu.VMEM((8, 16), jnp.float32),
                       pltpu.VMEM_SHARED((_N_ACC, 16), jnp.float32)],
    )
    def k(ewex_h, colp_h, out_h, idx_v, buf_v, zero_v, acc_sh):
        c = lax.axis_index("c")
        s = lax.axis_index("s")
        wid = c * _NS + s

        _zero_shared(acc_sh, zero_v, s, 16)
        plsc.subcore_barrier()

        def body(g, _):
            b = wid * _EPW + g * _G
            pltpu.sync_copy(colp_h.at[pl.ds(b, _G)], idx_v)
            pltpu.sync_copy(ewex_h.at[pl.ds(b, _G)], buf_v)
            pltpu.sync_copy(buf_v, acc_sh.at[idx_v], add=True)
            return _
        lax.fori_loop(0, _EPW // _G, body, 0)
        plsc.subcore_barrier()
        pltpu.sync_copy(acc_sh.at[pl.ds(s * _STRIPE, _STRIPE)],
                        out_h.at[c, pl.ds(s * _STRIPE, _STRIPE)])

    return k(ewex, colp)


def _bcast16(v, lane):
    idx = jnp.full((16, 1), lane, jnp.int32)
    dn = lax.GatherDimensionNumbers(offset_dims=(), collapsed_slice_dims=(0,),
                                    start_index_map=(0,))
    return lax.gather(v, idx, dn, slice_sizes=(1,),
                      mode=lax.GatherScatterMode.PROMISE_IN_BOUNDS)


def _zero_shared(acc_sh, zero_v, s, width):
    def zrow(i, _):
        for t in range(width // 16):
            zero_v[i, pl.ds(t * 16, 16)] = jnp.zeros((16,), jnp.float32)
        return _
    lax.fori_loop(0, 8, zrow, 0)

    def zcopy(j, _):
        pltpu.sync_copy(zero_v, acc_sh.at[pl.ds(s * _STRIPE + j * 8, 8)])
        return _
    lax.fori_loop(0, _STRIPE // 8, zcopy, 0)


_GB = 64


def _sc_passb(zg, xg, ewex, rowp, colp):
    @functools.partial(
        pl.kernel,
        out_type=(jax.ShapeDtypeStruct((_N_ACC, _HID), jnp.float32),
                  jax.ShapeDtypeStruct((_N_ACC, _HID), jnp.float32)),
        mesh=_mesh(),
        scratch_types=[pltpu.VMEM((_GB,), jnp.int32),
                       pltpu.VMEM((_GB,), jnp.int32),
                       pltpu.VMEM((_GB, _HID), jnp.float32),
                       pltpu.VMEM((_GB, 16), jnp.float32),
                       pltpu.VMEM((8, _HID), jnp.float32),
                       pltpu.VMEM_SHARED((_N_ACC, _HID), jnp.float32),
                       pltpu.SemaphoreType.DMA],
    )
    def k(zg_h, xg_h, ewex_h, rowp_h, colp_h, accg_h, acca_h,
          idxr_v, idxc_v, gbuf_v, wbuf_v, zero_v, acc_sh, sem):
        c = lax.axis_index("c")
        s = lax.axis_index("s")
        _zero_shared(acc_sh, zero_v, s, _HID)
        plsc.subcore_barrier()

        def run(tab_h, out_h, is_gcn):
            def body(g, _):
                b = s * (_E_PAD // _NS) + g * _GB
                pltpu.sync_copy(rowp_h.at[pl.ds(b, _GB)], idxr_v)
                pltpu.sync_copy(colp_h.at[pl.ds(b, _GB)], idxc_v)
                pltpu.async_copy(tab_h.at[idxr_v], gbuf_v, sem).wait()
                pltpu.sync_copy(ewex_h.at[pl.ds(b, _GB)], wbuf_v)

                def mul(e, _2):
                    wrow = wbuf_v[e, :]
                    if is_gcn:
                        w0 = _bcast16(wrow, 0)
                        for t in range(_HID // 16):
                            sl = pl.ds(t * 16, 16)
                            gbuf_v[e, sl] = gbuf_v[e, sl] * w0
                    else:
                        for h in range(_HEADS):
                            sl = pl.ds(h * 16, 16)
                            gbuf_v[e, sl] = gbuf_v[e, sl] * _bcast16(wrow, h + 1)
                    return _2
                lax.fori_loop(0, _GB, mul, 0)
                pltpu.sync_copy(gbuf_v, acc_sh.at[idxc_v], add=True)
                return _
            lax.fori_loop(0, _E_PAD // _NS // _GB, body, 0)
            plsc.subcore_barrier()
            pltpu.sync_copy(acc_sh.at[pl.ds(s * _STRIPE, _STRIPE)],
                            out_h.at[pl.ds(s * _STRIPE, _STRIPE)])

        @pl.when(c == 0)
        def _():
            run(zg_h, accg_h, True)

        @pl.when(c == 1)
        def _():
            run(xg_h, acca_h, False)

    return k(zg, xg, ewex, rowp, colp)


_RB = 1600


def _tc_input(xp, wT, consts):
    def body(x_r, w_r, c_r, o_r):
        t = jnp.dot(x_r[...], w_r[...],
                    preferred_element_type=jnp.float32) + c_r[0]
        o_r[...] = _gelu(_ln(t, c_r[1], c_r[2]))

    return pl.pallas_call(
        body,
        grid=(_N_ACC // _RB,),
        in_specs=[pl.BlockSpec((_RB, _D_IN), lambda i: (i, 0)),
                  pl.BlockSpec((_D_IN, _HID), lambda i: (0, 0)),
                  pl.BlockSpec((3, _HID), lambda i: (0, 0))],
        out_specs=pl.BlockSpec((_RB, _HID), lambda i: (i, 0)),
        out_shape=jax.ShapeDtypeStruct((_N_ACC, _HID), jnp.float32),
    )(xp, wT, consts)


def _tc_pre(xp, wcat, bias, lngb):
    def body(x_r, w_r, b_r, g_r, o_r, m_r):
        i = pl.program_id(0)
        h = _ln(x_r[...], g_r[0], g_r[1])
        t = jnp.dot(h, w_r[...], preferred_element_type=jnp.float32) + b_r[...]
        o_r[...] = t
        bm = jnp.max(t, axis=0, keepdims=True)
        bm8 = jnp.broadcast_to(bm, (8, 512))

        @pl.when(i == 0)
        def _():
            m_r[...] = bm8

        @pl.when(i > 0)
        def _():
            m_r[...] = jnp.maximum(m_r[...], bm8)

    return pl.pallas_call(
        body,
        grid=(_N_ACC // _RB,),
        in_specs=[pl.BlockSpec((_RB, _HID), lambda i: (i, 0)),
                  pl.BlockSpec((_HID, 512), lambda i: (0, 0)),
                  pl.BlockSpec((1, 512), lambda i: (0, 0)),
                  pl.BlockSpec((2, _HID), lambda i: (0, 0))],
        out_specs=[pl.BlockSpec((_RB, 512), lambda i: (i, 0)),
                   pl.BlockSpec((8, 512), lambda i: (0, 0))],
        out_shape=[jax.ShapeDtypeStruct((_N_ACC, 512), jnp.float32),
                   jax.ShapeDtypeStruct((8, 512), jnp.float32)],
    )(xp, wcat, bias, lngb)


_RBE = 1600


def _tc_edge(efS, efD, consts):
    def body(s_r, d_r, c_r, we_r):
        u = s_r[:, 0:64] + d_r[:, 0:64]
        g = _gelu(u)
        ew = jax.nn.sigmoid(jnp.sum(g * c_r[0, 0:64][None, :], axis=1,
                                    keepdims=True) + c_r[1, 0])
        alpha = s_r[:, 64:72] + d_r[:, 64:72]
        alpha = jnp.where(alpha >= 0, alpha, 0.2 * alpha)
        ex = jnp.exp(alpha - c_r[2, 0:8][None, :])
        we_r[...] = jnp.concatenate(
            [ew, ex, jnp.zeros((_RBE, 7), jnp.float32)], axis=1)

    return pl.pallas_call(
        body,
        grid=(_E_PAD // _RBE,),
        in_specs=[pl.BlockSpec((_RBE, _HID), lambda i: (i, 0)),
                  pl.BlockSpec((_RBE, _HID), lambda i: (i, 0)),
                  pl.BlockSpec((3, _HID), lambda i: (0, 0))],
        out_specs=pl.BlockSpec((_RBE, 16), lambda i: (i, 0)),
        out_shape=jax.ShapeDtypeStruct((_E_PAD, 16), jnp.float32),
    )(efS, efD, consts)


def _tc_mid(pretab, degden, mh_c):
    def body(p_r, dd_r, c_r, zg_r, aux_r):
        deg = dd_r[0, :, 0] + dd_r[1, :, 0] + 1.0
        dinv = lax.rsqrt(deg)[:, None]
        a_sd = p_r[:, 64:72] + p_r[:, 192:200]
        a_sd = jnp.where(a_sd >= 0, a_sd, 0.2 * a_sd)
        ex_self = jnp.exp(a_sd - c_r[0, 0:8][None, :])
        den = dd_r[0, :, 1:9] + dd_r[1, :, 1:9] + ex_self
        zg_r[...] = dinv * p_r[:, 256:384]
        z7 = jnp.zeros((_RB, 7), jnp.float32)
        z8 = jnp.zeros((_RB, 8), jnp.float32)
        aux_r[...] = jnp.concatenate([dinv, z7, den, ex_self, z8], axis=1)

    return pl.pallas_call(
        body,
        grid=(_N_ACC // _RB,),
        in_specs=[pl.BlockSpec((_RB, 512), lambda i: (i, 0)),
                  pl.BlockSpec((_NC, _RB, 16), lambda i: (0, i, 0)),
                  pl.BlockSpec((1, _HID), lambda i: (0, 0))],
        out_specs=[pl.BlockSpec((_RB, _HID), lambda i: (i, 0)),
                   pl.BlockSpec((_RB, 32), lambda i: (i, 0))],
        out_shape=[jax.ShapeDtypeStruct((_N_ACC, _HID), jnp.float32),
                   jax.ShapeDtypeStruct((_N_ACC, 32), jnp.float32)],
    )(pretab, degden, mh_c)


def _tc_post(x3, xwg3, xga3, aux3, accg3, acca3, gb, w1T, b1, w2T, cb2,
             se1, se2T):
    def body(x_r, xwg_r, xga_r, aux_r, ag_r, aa_r, gb_r, w1_r, b1_r, w2_r,
             c2_r, s1_r, s2_r, o_r):
        x = x_r[0]
        xwg = xwg_r[0]
        xga = xga_r[0]
        aux = aux_r[0]
        dinv = aux[:, 0:1]
        lane = lax.broadcasted_iota(jnp.int32, (8, _HID), 1) // _HD
        head = lax.broadcasted_iota(jnp.int32, (8, _HID), 0)
        rep = jnp.where(lane == head, 1.0, 0.0)
        den128 = jnp.dot(aux[:, 8:16], rep, preferred_element_type=jnp.float32)
        exs128 = jnp.dot(aux[:, 16:24], rep, preferred_element_type=jnp.float32)
        gcn = dinv * ag_r[0] + dinv * dinv * xwg
        gat = (aa_r[0] + exs128 * xga) / (den128 + 1e-16)
        x = gcn + gat + c2_r[2] + x
        h = _ln(x, gb_r[0], gb_r[1])
        t = _gelu(jnp.dot(h, w1_r[...], preferred_element_type=jnp.float32)
                  + b1_r[...])
        x = jnp.dot(t, w2_r[...], preferred_element_type=jnp.float32) \
            + c2_r[0:1] + x
        y = jnp.mean(x, axis=0, keepdims=True)
        t8 = _gelu(jnp.dot(y, s1_r[...].T,
                           preferred_element_type=jnp.float32))
        sc = jax.nn.sigmoid(jnp.dot(t8, s2_r[...],
                                    preferred_element_type=jnp.float32))
        o_r[0] = x * sc

    return pl.pallas_call(
        body,
        grid=(_B,),
        in_specs=[pl.BlockSpec((1, _S, _HID), lambda i: (i, 0, 0)),
                  pl.BlockSpec((1, _S, _HID), lambda i: (i, 0, 0)),
                  pl.BlockSpec((1, _S, _HID), lambda i: (i, 0, 0)),
                  pl.BlockSpec((1, _S, 32), lambda i: (i, 0, 0)),
                  pl.BlockSpec((1, _S, _HID), lambda i: (i, 0, 0)),
                  pl.BlockSpec((1, _S, _HID), lambda i: (i, 0, 0)),
                  pl.BlockSpec((2, _HID), lambda i: (0, 0)),
                  pl.BlockSpec((_HID, 512), lambda i: (0, 0)),
                  pl.BlockSpec((1, 512), lambda i: (0, 0)),
                  pl.BlockSpec((512, _HID), lambda i: (0, 0)),
                  pl.BlockSpec((3, _HID), lambda i: (0, 0)),
                  pl.BlockSpec((8, _HID), lambda i: (0, 0)),
                  pl.BlockSpec((8, _HID), lambda i: (0, 0))],
        out_specs=pl.BlockSpec((1, _S, _HID), lambda i: (i, 0, 0)),
        out_shape=jax.ShapeDtypeStruct((_B, _S, _HID), jnp.float32),
    )(x3, xwg3, xga3, aux3, accg3, acca3, gb, w1T, b1, w2T, cb2, se1, se2T)


def _tc_final(x3, w6, wblk, fusT, consts):
    def body(x_r, w6_r, wb_r, f_r, c_r, o_r):
        xb = x_r[0]
        pooled = jnp.dot(w6_r[...], xb,
                         preferred_element_type=jnp.float32)
        pyr = c_r[3:4]
        for i in range(4):
            pyr = pyr + jnp.dot(pooled[i:i + 1], wb_r[i],
                                preferred_element_type=jnp.float32)
        mean = pooled[4:5]
        mx = jnp.max(xb, axis=0, keepdims=True)
        cat = jnp.dot(pyr, f_r[0], preferred_element_type=jnp.float32) \
            + jnp.dot(mean, f_r[1], preferred_element_type=jnp.float32) \
            + jnp.dot(mx, f_r[2], preferred_element_type=jnp.float32) \
            + c_r[0:1]
        o_r[0] = _gelu(_ln(cat, c_r[1], c_r[2]))

    return pl.pallas_call(
        body,
        grid=(_B,),
        in_specs=[pl.BlockSpec((1, _S, _HID), lambda i: (i, 0, 0)),
                  pl.BlockSpec((8, _S), lambda i: (0, 0)),
                  pl.BlockSpec((4, _HID, _HID), lambda i: (0, 0, 0)),
                  pl.BlockSpec((3, _HID, _HID), lambda i: (0, 0, 0)),
                  pl.BlockSpec((4, _HID), lambda i: (0, 0))],
        out_specs=pl.BlockSpec((1, 1, _HID), lambda i: (i, 0, 0)),
        out_shape=jax.ShapeDtypeStruct((_B, 1, _HID), jnp.float32),
    )(x3, w6, wblk, fusT, consts)


def _pool_weights(levels=(1, 2, 4, 8), s=_S):
    ws = []
    for lvl in levels:
        ps = s // lvl
        w = np.zeros((s,), np.float64)
        for i in range(ps):
            st = (i * s) // ps
            e = -(-((i + 1) * s) // ps)
            w[st:e] += 1.0 / (e - st) / ps
        ws.append(w)
    return np.stack(ws, 0).astype(np.float32)


def kernel(x, params, edge_index):
    p = params
    f32 = jnp.float32
    row = edge_index[0].astype(jnp.int32)
    col = edge_index[1].astype(jnp.int32)
    rowp = jnp.concatenate([row, jnp.full((_E_PAD - _E,), _N, jnp.int32)])
    colp = jnp.concatenate([col, jnp.full((_E_PAD - _E,), _N, jnp.int32)])

    xp = jnp.pad(x, ((0, _N_ACC - _N), (0, 0)))
    consts_in = jnp.stack([p['in_b'], p['in_g'], p['in_bt']], 0)
    xcur = _tc_input(xp, p['in_w'].T, consts_in)

    for lp in p['layers']:
        gatT = lp['gat_w'].T
        a_s_w = (gatT.reshape(_HID, _HEADS, _HD)
                 * lp['gat_as'][None]).sum(-1)
        a_d_w = (gatT.reshape(_HID, _HEADS, _HD)
                 * lp['gat_ad'][None]).sum(-1)
        zc = jnp.zeros((_HID, 56), f32)
        wcat = jnp.concatenate([
            lp['em_w1'][:, :_HID].T, a_s_w, zc,
            lp['em_w1'][:, _HID:].T, a_d_w, zc,
            lp['gcn_w'].T, gatT], axis=1)
        bias = jnp.zeros((512,), f32).at[0:64].set(lp['em_b1'])[None, :]
        lngb = jnp.stack([lp['pre_g'], lp['pre_b']], 0)

        pretab, colmax = _tc_pre(xcur, wcat, bias, lngb)
        mh = colmax[0, 64:72] + colmax[0, 192:200]

        tabS = pretab[:, 0:128]
        tabD = pretab[:, 128:256]
        efS, efD = _sc_gather(tabS, tabD, rowp, colp)

        consts_e = jnp.zeros((3, _HID), f32)
        consts_e = consts_e.at[0, 0:64].set(lp['em_w2'][0])
        consts_e = consts_e.at[1, 0].set(lp['em_b2'][0])
        consts_e = consts_e.at[2, 0:8].set(mh)
        ewex = _tc_edge(efS, efD, consts_e)

        degden = _sc_degden(ewex, colp)
        mh_c = jnp.zeros((1, _HID), f32).at[0, 0:8].set(mh)
        zg, aux = _tc_mid(pretab, degden, mh_c)

        xgtab = pretab[:, 384:512]
        accg, acca = _sc_passb(zg, xgtab, ewex, rowp, colp)

        x3 = xcur[:_N].reshape(_B, _S, _HID)
        xwg3 = pretab[:_N, 256:384].reshape(_B, _S, _HID)
        xga3 = pretab[:_N, 384:512].reshape(_B, _S, _HID)
        aux3 = aux[:_N].reshape(_B, _S, 32)
        accg3 = accg[:_N].reshape(_B, _S, _HID)
        acca3 = acca[:_N].reshape(_B, _S, _HID)
        gb = jnp.stack([lp['post_g'], lp['post_b']], 0)
        cb2 = jnp.stack([lp['ffn_b2'],
                         jnp.zeros((_HID,), f32),
                         lp['gcn_b'] + lp['gat_b']], 0)
        xg3 = _tc_post(x3, xwg3, xga3, aux3, accg3, acca3, gb,
                       lp['ffn_w1'].T, lp['ffn_b1'][None, :], lp['ffn_w2'].T,
                       cb2, lp['se_w1'], lp['se_w2'].T)
        xcur = jnp.pad(xg3.reshape(_N, _HID), ((0, _N_ACC - _N), (0, 0)))

    x3 = xcur[:_N].reshape(_B, _S, _HID)
    w6 = jnp.zeros((8, _S), f32)
    w6 = w6.at[0:4].set(jnp.asarray(_pool_weights()))
    w6 = w6.at[4].set(jnp.full((_S,), 1.0 / _S, f32))
    wblk = jnp.zeros((4, _HID, _HID), f32)
    for i in range(4):
        wblk = wblk.at[i, :, 32 * i:32 * (i + 1)].set(p['pp'][i][0].T)
    ppb = jnp.concatenate([p['pp'][i][1] for i in range(4)])
    fusT = p['fus_w'].T.reshape(3, _HID, _HID)
    consts_f = jnp.stack([p['fus_b'], p['fus_g'], p['fus_bt'], ppb], 0)
    return _tc_final(x3, w6, wblk, fusT, consts_f).reshape(_B, _HID)
